# Initial kernel scaffold; baseline (speedup 1.0000x reference)
import jax
import jax.numpy as jnp
from jax.experimental import pallas as pl


def kernel(target_tokens, context_tokens, neg_idx, W_fwd, W_rev, W_iso, token_frequencies):
    raise NotImplementedError("write your pallas kernel here")


# speedup vs baseline: 9.5769x; 9.5769x over previous
import functools

import jax
import jax.numpy as jnp
from jax import lax
from jax.experimental import pallas as pl
from jax.experimental.pallas import tpu as pltpu
from jax.experimental.pallas import tpu_sc as plsc

V = 100000
D = 64
B = 16384
L = 50
K = 5
ALPHA = 0.5

NC = 2
NS = 16
NW = NC * NS
BPW = B // NW
CB = 16
NCHUNK = BPW // CB

LP = 64
KP = 16


def _bcast_lane(vec, lane):
    idx = jnp.full((16, 1), lane, dtype=jnp.int32)
    dnums = lax.GatherDimensionNumbers(
        offset_dims=(), collapsed_slice_dims=(0,), start_index_map=(0,))
    return lax.gather(vec, idx, dnums, slice_sizes=(1,),
                      mode=lax.GatherScatterMode.PROMISE_IN_BOUNDS)


def _sc_body(tgt_ref, ctx_ref, negi_ref, wf_ref, wr_ref, wiso_ref, scal_ref,
             pos_out, neg_out,
             tgt_v, ctx_v, negi_v, fwd_v, rev_v, iso_v, scal_v,
             ctx_rows, neg_rows, pos_sv, neg_sv, sem):
    wid = lax.axis_index("s") * NC + lax.axis_index("c")
    row0 = wid * BPW

    def chunk_body(ci, carry):
        base = row0 + ci * CB

        pltpu.sync_copy(tgt_ref.at[pl.ds(base, CB)], tgt_v)
        pltpu.sync_copy(ctx_ref.at[pl.ds(base * L, CB * L)], ctx_v)
        pltpu.sync_copy(negi_ref.at[pl.ds(base * K, CB * K)], negi_v)

        cps = [
            pltpu.async_copy(wf_ref.at[tgt_v], fwd_v, sem),
            pltpu.async_copy(wr_ref.at[tgt_v], rev_v, sem),
            pltpu.async_copy(wiso_ref.at[tgt_v], iso_v, sem),
            pltpu.async_copy(scal_ref.at[tgt_v], scal_v, sem),
            pltpu.async_copy(wf_ref.at[ctx_v], ctx_rows, sem),
            pltpu.async_copy(wf_ref.at[negi_v], neg_rows, sem),
        ]
        for cp in cps:
            cp.wait()

        sc_all = scal_v[pl.ds(0, CB)]
        lanes = jnp.arange(16, dtype=jnp.int32)

        def b_body(b, _):
            sc = _bcast_lane(sc_all, b)
            og = []
            for g in range(4):
                f = fwd_v[b, pl.ds(g * 16, 16)]
                r = rev_v[b, pl.ds(g * 16, 16)]
                s = iso_v[b, pl.ds(g * 16, 16)]
                og.append(f * ALPHA + r * (1.0 - ALPHA) + s * sc)

            def dot(rows_ref, row):
                p = og[0] * rows_ref[row, pl.ds(0, 16)]
                for g in range(1, 4):
                    p = p + og[g] * rows_ref[row, pl.ds(g * 16, 16)]
                cum = plsc.cumsum(p)
                return _bcast_lane(cum, 15)

            for gl in range(4):
                nl = min(16, L - gl * 16)
                sv = jnp.zeros((16,), jnp.float32)
                for j in range(nl):
                    s = dot(ctx_rows, b * L + gl * 16 + j)
                    sv = jnp.where(lanes == j, s, sv)
                pos_sv[b, pl.ds(gl * 16, 16)] = sv

            sv = jnp.zeros((16,), jnp.float32)
            for k in range(K):
                s = dot(neg_rows, b * K + k)
                sv = jnp.where(lanes == k, s, sv)
            neg_sv[b, pl.ds(0, 16)] = sv
            return _

        lax.fori_loop(0, CB, b_body, None)

        pltpu.sync_copy(pos_sv, pos_out.at[pl.ds(base, CB)])
        pltpu.sync_copy(neg_sv, neg_out.at[pl.ds(base, CB)])
        return carry

    lax.fori_loop(0, NCHUNK, chunk_body, None)


_sc_scores = functools.partial(
    pl.kernel,
    out_type=(
        jax.ShapeDtypeStruct((B, LP), jnp.float32),
        jax.ShapeDtypeStruct((B, KP), jnp.float32),
    ),
    mesh=plsc.VectorSubcoreMesh(
        core_axis_name="c", subcore_axis_name="s",
        num_cores=NC, num_subcores=NS),
    compiler_params=pltpu.CompilerParams(
        needs_layout_passes=False, use_tc_tiling_on_sc=False),
    scratch_types=[
        pltpu.VMEM((CB,), jnp.int32),
        pltpu.VMEM((CB * L,), jnp.int32),
        pltpu.VMEM((CB * K,), jnp.int32),
        pltpu.VMEM((CB, D), jnp.float32),
        pltpu.VMEM((CB, D), jnp.float32),
        pltpu.VMEM((CB, D), jnp.float32),
        pltpu.VMEM((CB,), jnp.float32),
        pltpu.VMEM((CB * L, D), jnp.float32),
        pltpu.VMEM((CB * K, D), jnp.float32),
        pltpu.VMEM((CB, LP), jnp.float32),
        pltpu.VMEM((CB, KP), jnp.float32),
        pltpu.SemaphoreType.DMA,
    ],
)(_sc_body)


def _loss_body(pos_ref, neg_ref, out_ref):
    pos = pos_ref[...]
    lane_p = lax.broadcasted_iota(jnp.int32, (B, LP), 1)
    pt = -jnp.log(jax.nn.sigmoid(pos) + 1e-6)
    psum = jnp.sum(jnp.where(lane_p < L, pt, 0.0))

    neg = neg_ref[...]
    lane_n = lax.broadcasted_iota(jnp.int32, (B, KP), 1)
    nt = -jnp.log(jax.nn.sigmoid(-neg) + 1e-6)
    nsum = jnp.sum(jnp.where(lane_n < K, nt, 0.0))

    out_ref[0, 0] = psum / (B * float(L)) + nsum / float(K)


_loss_tc = pl.pallas_call(
    _loss_body,
    out_shape=jax.ShapeDtypeStruct((1, 1), jnp.float32),
    out_specs=pl.BlockSpec(memory_space=pltpu.SMEM),
)


def kernel(target_tokens, context_tokens, neg_idx, W_fwd, W_rev, W_iso,
           token_frequencies):
    tgt = target_tokens.astype(jnp.int32)
    ctx = context_tokens.astype(jnp.int32).reshape(B * L)
    neg = neg_idx.astype(jnp.int32).reshape(B * K)
    scal = 1.0 / (1.0 + jnp.log(token_frequencies + 1e-6))
    pos_s, neg_s = _sc_scores(tgt, ctx, neg, W_fwd, W_rev, W_iso, scal)
    return _loss_tc(pos_s, neg_s)[0, 0]
